# baseline (speedup 1.0000x reference)
import functools

import jax
import jax.numpy as jnp
from jax import lax
from jax.experimental import pallas as pl
from jax.experimental.pallas import tpu as pltpu
from jax.experimental.pallas import tpu_sc as plsc

T = 2048
D = 1024
DH = D // 2
E = 8
DF = 512
CAP = 512
S = E * CAP
NW = 32
SPT = S // NW
TPT = T // NW
L = 16
_NEG = -1e30


def _routing_body(x_ref, gw_ref, ri_ref, rf_ref, xp_ref):
    x = x_ref[...]
    gw = gw_ref[...]
    xp_ref[...] = _pack_rows(x)
    logits = jax.lax.dot_general(
        x, gw, (((1,), (1,)), ((), ())), preferred_element_type=jnp.float32
    )
    m = jnp.max(logits, axis=-1, keepdims=True)
    ex = jnp.exp(logits - m)
    scores = ex / jnp.sum(ex, axis=-1, keepdims=True)
    eidx = jax.lax.broadcasted_iota(jnp.int32, (T, E), 1)
    s0 = jnp.max(scores, axis=-1, keepdims=True)
    i0 = jnp.min(jnp.where(scores >= s0, eidx, E), axis=-1, keepdims=True)
    oh0 = eidx == i0
    sc1 = jnp.where(oh0, _NEG, scores)
    s1 = jnp.max(sc1, axis=-1, keepdims=True)
    i1 = jnp.min(jnp.where(sc1 >= s1, eidx, E), axis=-1, keepdims=True)
    oh1 = eidx == i1
    cnt = oh0.astype(jnp.bfloat16) + oh1.astype(jnp.bfloat16)
    nb = 8
    tb = T // nb
    r = jax.lax.broadcasted_iota(jnp.int32, (tb, tb), 0)
    c = jax.lax.broadcasted_iota(jnp.int32, (tb, tb), 1)
    lmask = (c < r).astype(jnp.bfloat16)
    ones = jnp.ones((1, tb), jnp.bfloat16)
    parts = []
    run = jnp.zeros((1, E), jnp.float32)
    for b in range(nb):
        cb = cnt[b * tb:(b + 1) * tb, :]
        intra = jax.lax.dot_general(
            lmask, cb, (((1,), (0,)), ((), ())), preferred_element_type=jnp.float32
        )
        parts.append(intra + run)
        tot = jax.lax.dot_general(
            ones, cb, (((1,), (0,)), ((), ())), preferred_element_type=jnp.float32
        )
        run = run + tot
    cum = jnp.concatenate(parts, axis=0)
    pos0 = jnp.sum(jnp.where(oh0, cum, 0.0), axis=-1, keepdims=True).astype(jnp.int32)
    pos1 = jnp.sum(jnp.where(oh1, cum, 0.0), axis=-1, keepdims=True).astype(jnp.int32)
    v0 = pos0 < CAP
    v1 = pos1 < CAP
    denom = s0 + s1 + 1e-20
    w0 = jnp.where(v0, s0 / denom, 0.0)
    w1 = jnp.where(v1, s1 / denom, 0.0)
    d0 = i0 * CAP + pos0
    d1 = i1 * CAP + pos1
    d0m = jnp.where(v0, d0, -1)
    d1m = jnp.where(v1, d1, -1)
    d0c = jnp.where(v0, d0, 0)
    d1c = jnp.where(v1, d1, 0)
    zi = jnp.zeros((T, 1), jnp.int32)
    ri = jnp.concatenate([d0m, d1m, d0c, d1c, zi, zi, zi, zi], axis=1)
    zf = jnp.zeros((T, 1), jnp.float32)
    rf = jnp.concatenate([w0, w1, zf, zf, zf, zf, zf, zf], axis=1)
    ri_ref[...] = ri
    rf_ref[...] = rf


def _routing(x, gate_w):
    return pl.pallas_call(
        _routing_body,
        out_shape=(
            jax.ShapeDtypeStruct((T, E), jnp.int32),
            jax.ShapeDtypeStruct((T, E), jnp.float32),
            jax.ShapeDtypeStruct((T, DH), jnp.int32),
        ),
    )(x, gate_w)


def _dispatch_body(ri_hbm, xp_hbm, xg_hbm, riv, src, rows, sem):
    wid = lax.axis_index("s") * 2 + lax.axis_index("c")
    base = wid * SPT
    pltpu.sync_copy(ri_hbm, riv)
    for k in range(SPT // L):
        src[pl.ds(k * L, L)] = jnp.zeros((L,), jnp.int32)
    lanes = jax.lax.broadcasted_iota(jnp.int32, (L,), 0)

    def chunk(c, carry):
        rowi = c * L + lanes
        flat = rowi * E
        d0 = plsc.load_gather(riv, [flat])
        d1 = plsc.load_gather(riv, [flat + 1])
        m0 = (d0 >= base) & (d0 < base + SPT)
        l0 = jnp.where(m0, d0 - base, 0)
        plsc.store_scatter(src, [l0], rowi, mask=m0)
        m1 = (d1 >= base) & (d1 < base + SPT)
        l1 = jnp.where(m1, d1 - base, 0)
        plsc.store_scatter(src, [l1], rowi, mask=m1)
        return carry

    lax.fori_loop(0, T // L, chunk, 0)
    half = SPT // 2
    for g in range(2):
        pltpu.async_copy(xp_hbm.at[src.at[pl.ds(g * half, half)]], rows, sem).wait()
        pltpu.sync_copy(rows, xg_hbm.at[pl.ds(base + g * half, half)])


def _dispatch(ri, xp):
    mesh = plsc.VectorSubcoreMesh(core_axis_name="c", subcore_axis_name="s")
    half = SPT // 2
    f = pl.kernel(
        _dispatch_body,
        out_type=jax.ShapeDtypeStruct((S, DH), jnp.int32),
        mesh=mesh,
        scratch_types=[
            pltpu.VMEM((T * E,), jnp.int32),
            pltpu.VMEM((SPT,), jnp.int32),
            pltpu.VMEM((half, DH), jnp.int32),
            pltpu.SemaphoreType.DMA,
        ],
        compiler_params=pltpu.CompilerParams(needs_layout_passes=False),
    )
    return f(ri.reshape(T * E), xp)


def _silu(h):
    return h / (1.0 + jnp.exp(-h))


def _pack_rows(r):
    rb = r.astype(jnp.bfloat16).astype(jnp.float32)
    ub = jax.lax.bitcast_convert_type(rb, jnp.uint32)
    lo = ub[:, :DH] >> 16
    hi = ub[:, DH:] & jnp.uint32(0xFFFF0000)
    return jax.lax.bitcast_convert_type(hi | lo, jnp.int32)


def _unpack_rows(p):
    ub = jax.lax.bitcast_convert_type(p, jnp.uint32)
    lo = jax.lax.bitcast_convert_type(ub << 16, jnp.float32)
    hi = jax.lax.bitcast_convert_type(ub & jnp.uint32(0xFFFF0000), jnp.float32)
    return jnp.concatenate([lo, hi], axis=1).astype(jnp.bfloat16)


def _experts_body(xg_ref, w1_ref, w3_ref, w2_ref, eo_ref):
    xgb = _unpack_rows(xg_ref[0])
    a = w1_ref[0].astype(jnp.bfloat16)
    b = w3_ref[0].astype(jnp.bfloat16)
    cw = w2_ref[0].astype(jnp.bfloat16)
    h1 = jax.lax.dot_general(
        xgb, a, (((1,), (1,)), ((), ())), preferred_element_type=jnp.float32
    )
    h3 = jax.lax.dot_general(
        xgb, b, (((1,), (1,)), ((), ())), preferred_element_type=jnp.float32
    )
    h = (_silu(h1) * h3).astype(jnp.bfloat16)
    out = jax.lax.dot_general(
        h, cw, (((1,), (1,)), ((), ())), preferred_element_type=jnp.float32
    )
    eo_ref[0] = _pack_rows(out)


def _experts(xg, w1, w3, w2):
    return pl.pallas_call(
        _experts_body,
        grid=(E,),
        in_specs=[
            pl.BlockSpec((1, CAP, DH), lambda e: (e, 0, 0)),
            pl.BlockSpec((1, DF, D), lambda e: (e, 0, 0)),
            pl.BlockSpec((1, DF, D), lambda e: (e, 0, 0)),
            pl.BlockSpec((1, D, DF), lambda e: (e, 0, 0)),
        ],
        out_specs=pl.BlockSpec((1, CAP, DH), lambda e: (e, 0, 0)),
        out_shape=jax.ShapeDtypeStruct((E, CAP, DH), jnp.int32),
        compiler_params=pltpu.CompilerParams(
            dimension_semantics=("arbitrary",),
        ),
    )(xg.reshape(E, CAP, DH), w1, w3, w2)


_SB = 8


def _shared_body(xp_ref, sw1_ref, sw3_ref, sw2_ref, sh_ref):
    xs = _unpack_rows(xp_ref[...])
    sa = sw1_ref[...].astype(jnp.bfloat16)
    sb = sw3_ref[...].astype(jnp.bfloat16)
    sc = sw2_ref[...].astype(jnp.bfloat16)
    sh1 = jax.lax.dot_general(
        xs, sa, (((1,), (1,)), ((), ())), preferred_element_type=jnp.float32
    )
    sh3 = jax.lax.dot_general(
        xs, sb, (((1,), (1,)), ((), ())), preferred_element_type=jnp.float32
    )
    hs = (_silu(sh1) * sh3).astype(jnp.bfloat16)
    out = jax.lax.dot_general(
        hs, sc, (((1,), (1,)), ((), ())), preferred_element_type=jnp.float32
    )
    sh_ref[...] = _pack_rows(out)


def _shared(xp, sw1, sw3, sw2):
    tb = T // _SB
    return pl.pallas_call(
        _shared_body,
        grid=(_SB,),
        in_specs=[
            pl.BlockSpec((tb, DH), lambda i: (i, 0)),
            pl.BlockSpec((2 * DF, D), lambda i: (0, 0)),
            pl.BlockSpec((2 * DF, D), lambda i: (0, 0)),
            pl.BlockSpec((D, 2 * DF), lambda i: (0, 0)),
        ],
        out_specs=pl.BlockSpec((tb, DH), lambda i: (i, 0)),
        out_shape=jax.ShapeDtypeStruct((T, DH), jnp.int32),
        compiler_params=pltpu.CompilerParams(
            dimension_semantics=("arbitrary",),
        ),
    )(xp, sw1, sw3, sw2)


_CHT = 32


def _combine_body(ri_hbm, rf_hbm, eo_hbm, sh_hbm, y_hbm,
                  riv, rfv, idx0, idx1, g0, g1, shb, yb, sem0, sem1, sem2):
    wid = lax.axis_index("s") * 2 + lax.axis_index("c")
    tb = wid * TPT
    pltpu.sync_copy(ri_hbm.at[pl.ds(tb * E, TPT * E)], riv)
    pltpu.sync_copy(rf_hbm.at[pl.ds(tb * E, TPT * E)], rfv)
    lanes = jax.lax.broadcasted_iota(jnp.int32, (L,), 0)
    for h in range(TPT // _CHT):
        for c in range(_CHT // L):
            flat = (h * _CHT + c * L + lanes) * E
            idx0[pl.ds(c * L, L)] = plsc.load_gather(riv, [flat + 2])
            idx1[pl.ds(c * L, L)] = plsc.load_gather(riv, [flat + 3])
        cp0 = pltpu.async_copy(eo_hbm.at[idx0], g0, sem0)
        cp1 = pltpu.async_copy(eo_hbm.at[idx1], g1, sem1)
        cp2 = pltpu.async_copy(sh_hbm.at[pl.ds(tb + h * _CHT, _CHT)], shb, sem2)
        cp0.wait()
        cp1.wait()
        cp2.wait()

        def tok(t, carry):
            row = jnp.full((L,), (h * _CHT + t) * E, jnp.int32)
            w0 = plsc.load_gather(rfv, [row])
            w1 = plsc.load_gather(rfv, [row + 1])
            w0b = plsc.pack(w0, w0, format=plsc.PackFormat.INTERLEAVED)
            w1b = plsc.pack(w1, w1, format=plsc.PackFormat.INTERLEAVED)
            for j in range(DH // L):
                va = plsc.bitcast(g0[t, pl.ds(j * L, L)], jnp.bfloat16)
                vb = plsc.bitcast(g1[t, pl.ds(j * L, L)], jnp.bfloat16)
                vs = plsc.bitcast(shb[t, pl.ds(j * L, L)], jnp.bfloat16)
                acc = vs + w0b * va + w1b * vb
                alo, ahi = plsc.unpack(acc, format=plsc.PackFormat.INTERLEAVED)
                yb[t, pl.ds(j * L, L)] = alo
                yb[t, pl.ds(DH + j * L, L)] = ahi
            return carry

        lax.fori_loop(0, _CHT, tok, 0)
        pltpu.sync_copy(yb, y_hbm.at[pl.ds(tb + h * _CHT, _CHT)])


def _combine(ri, rf, eo, sh):
    mesh = plsc.VectorSubcoreMesh(core_axis_name="c", subcore_axis_name="s")
    f = pl.kernel(
        _combine_body,
        out_type=jax.ShapeDtypeStruct((T, D), jnp.float32),
        mesh=mesh,
        scratch_types=[
            pltpu.VMEM((TPT * E,), jnp.int32),
            pltpu.VMEM((TPT * E,), jnp.float32),
            pltpu.VMEM((_CHT,), jnp.int32),
            pltpu.VMEM((_CHT,), jnp.int32),
            pltpu.VMEM((_CHT, DH), jnp.int32),
            pltpu.VMEM((_CHT, DH), jnp.int32),
            pltpu.VMEM((_CHT, DH), jnp.int32),
            pltpu.VMEM((_CHT, D), jnp.float32),
            pltpu.SemaphoreType.DMA,
            pltpu.SemaphoreType.DMA,
            pltpu.SemaphoreType.DMA,
        ],
        compiler_params=pltpu.CompilerParams(needs_layout_passes=False),
    )
    return f(ri.reshape(T * E), rf.reshape(T * E), eo, sh)


def kernel(hidden_states, gate_w, w1, w3, w2, sw1, sw3, sw2):
    orig_shape = hidden_states.shape
    x = hidden_states.reshape(T, D).astype(jnp.float32)
    ri, rf, xp = _routing(x, gate_w.astype(jnp.float32))
    xg = _dispatch(ri, xp)
    eo = _experts(xg, w1, w3, w2)
    sh = _shared(xp, sw1, sw3, sw2)
    y = _combine(ri, rf, eo.reshape(S, DH), sh)
    return y.reshape(orig_shape)


# speedup vs baseline: 1.0713x; 1.0713x over previous
import functools

import jax
import jax.numpy as jnp
from jax import lax
from jax.experimental import pallas as pl
from jax.experimental.pallas import tpu as pltpu
from jax.experimental.pallas import tpu_sc as plsc

T = 2048
D = 1024
DH = D // 2
E = 8
DF = 512
CAP = 512
S = E * CAP
NW = 32
SPT = S // NW
TPT = T // NW
L = 16
_NEG = -1e30


def _routing_body(x_ref, gw_ref, ri_ref, rf_ref, xp_ref):
    x = x_ref[...]
    gw = gw_ref[...]
    xp_ref[...] = _pack_rows(x)
    logits = jax.lax.dot_general(
        x, gw, (((1,), (1,)), ((), ())), preferred_element_type=jnp.float32
    )
    m = jnp.max(logits, axis=-1, keepdims=True)
    ex = jnp.exp(logits - m)
    scores = ex / jnp.sum(ex, axis=-1, keepdims=True)
    eidx = jax.lax.broadcasted_iota(jnp.int32, (T, E), 1)
    s0 = jnp.max(scores, axis=-1, keepdims=True)
    i0 = jnp.min(jnp.where(scores >= s0, eidx, E), axis=-1, keepdims=True)
    oh0 = eidx == i0
    sc1 = jnp.where(oh0, _NEG, scores)
    s1 = jnp.max(sc1, axis=-1, keepdims=True)
    i1 = jnp.min(jnp.where(sc1 >= s1, eidx, E), axis=-1, keepdims=True)
    oh1 = eidx == i1
    cnt = oh0.astype(jnp.bfloat16) + oh1.astype(jnp.bfloat16)
    nb = 8
    tb = T // nb
    r = jax.lax.broadcasted_iota(jnp.int32, (tb, tb), 0)
    c = jax.lax.broadcasted_iota(jnp.int32, (tb, tb), 1)
    lmask = (c < r).astype(jnp.bfloat16)
    ones = jnp.ones((1, tb), jnp.bfloat16)
    parts = []
    run = jnp.zeros((1, E), jnp.float32)
    for b in range(nb):
        cb = cnt[b * tb:(b + 1) * tb, :]
        intra = jax.lax.dot_general(
            lmask, cb, (((1,), (0,)), ((), ())), preferred_element_type=jnp.float32
        )
        parts.append(intra + run)
        tot = jax.lax.dot_general(
            ones, cb, (((1,), (0,)), ((), ())), preferred_element_type=jnp.float32
        )
        run = run + tot
    cum = jnp.concatenate(parts, axis=0)
    pos0 = jnp.sum(jnp.where(oh0, cum, 0.0), axis=-1, keepdims=True).astype(jnp.int32)
    pos1 = jnp.sum(jnp.where(oh1, cum, 0.0), axis=-1, keepdims=True).astype(jnp.int32)
    v0 = pos0 < CAP
    v1 = pos1 < CAP
    denom = s0 + s1 + 1e-20
    w0 = jnp.where(v0, s0 / denom, 0.0)
    w1 = jnp.where(v1, s1 / denom, 0.0)
    d0 = i0 * CAP + pos0
    d1 = i1 * CAP + pos1
    d0m = jnp.where(v0, d0, -1)
    d1m = jnp.where(v1, d1, -1)
    d0c = jnp.where(v0, d0, 0)
    d1c = jnp.where(v1, d1, 0)
    zi = jnp.zeros((T, 1), jnp.int32)
    ri = jnp.concatenate([d0m, d1m, d0c, d1c, zi, zi, zi, zi], axis=1)
    zf = jnp.zeros((T, 1), jnp.float32)
    rf = jnp.concatenate([w0, w1, zf, zf, zf, zf, zf, zf], axis=1)
    ri_ref[...] = ri
    rf_ref[...] = rf


def _routing(x, gate_w):
    return pl.pallas_call(
        _routing_body,
        out_shape=(
            jax.ShapeDtypeStruct((T, E), jnp.int32),
            jax.ShapeDtypeStruct((T, E), jnp.float32),
            jax.ShapeDtypeStruct((T, DH), jnp.int32),
        ),
    )(x, gate_w)


def _dispatch_body(ri_hbm, xp_hbm, xg_hbm, riv, src, rows, sem):
    wid = lax.axis_index("s") * 2 + lax.axis_index("c")
    base = wid * SPT
    pltpu.sync_copy(ri_hbm, riv)
    for k in range(SPT // L):
        src[pl.ds(k * L, L)] = jnp.zeros((L,), jnp.int32)
    lanes = jax.lax.broadcasted_iota(jnp.int32, (L,), 0)

    def chunk(c, carry):
        rowi = c * L + lanes
        flat = rowi * E
        d0 = plsc.load_gather(riv, [flat])
        d1 = plsc.load_gather(riv, [flat + 1])
        m0 = (d0 >= base) & (d0 < base + SPT)
        l0 = jnp.where(m0, d0 - base, 0)
        plsc.store_scatter(src, [l0], rowi, mask=m0)
        m1 = (d1 >= base) & (d1 < base + SPT)
        l1 = jnp.where(m1, d1 - base, 0)
        plsc.store_scatter(src, [l1], rowi, mask=m1)
        return carry

    lax.fori_loop(0, T // L, chunk, 0)
    half = SPT // 2
    for g in range(2):
        pltpu.async_copy(xp_hbm.at[src.at[pl.ds(g * half, half)]], rows, sem).wait()
        pltpu.sync_copy(rows, xg_hbm.at[pl.ds(base + g * half, half)])


def _dispatch(ri, xp):
    mesh = plsc.VectorSubcoreMesh(core_axis_name="c", subcore_axis_name="s")
    half = SPT // 2
    f = pl.kernel(
        _dispatch_body,
        out_type=jax.ShapeDtypeStruct((S, DH), jnp.int32),
        mesh=mesh,
        scratch_types=[
            pltpu.VMEM((T * E,), jnp.int32),
            pltpu.VMEM((SPT,), jnp.int32),
            pltpu.VMEM((half, DH), jnp.int32),
            pltpu.SemaphoreType.DMA,
        ],
        compiler_params=pltpu.CompilerParams(needs_layout_passes=False),
    )
    return f(ri.reshape(T * E), xp)


def _silu(h):
    return h / (1.0 + jnp.exp(-h))


def _pack_rows(r):
    rb = r.astype(jnp.bfloat16).astype(jnp.float32)
    ub = jax.lax.bitcast_convert_type(rb, jnp.uint32)
    lo = ub[:, :DH] >> 16
    hi = ub[:, DH:] & jnp.uint32(0xFFFF0000)
    return jax.lax.bitcast_convert_type(hi | lo, jnp.int32)


def _unpack_rows(p):
    ub = jax.lax.bitcast_convert_type(p, jnp.uint32)
    lo = jax.lax.bitcast_convert_type(ub << 16, jnp.float32)
    hi = jax.lax.bitcast_convert_type(ub & jnp.uint32(0xFFFF0000), jnp.float32)
    return jnp.concatenate([lo, hi], axis=1).astype(jnp.bfloat16)


def _experts_body(xg_ref, w1_ref, w3_ref, w2_ref, eo_ref):
    xgb = _unpack_rows(xg_ref[0])
    a = w1_ref[0].astype(jnp.bfloat16)
    b = w3_ref[0].astype(jnp.bfloat16)
    cw = w2_ref[0].astype(jnp.bfloat16)
    h1 = jax.lax.dot_general(
        xgb, a, (((1,), (1,)), ((), ())), preferred_element_type=jnp.float32
    )
    h3 = jax.lax.dot_general(
        xgb, b, (((1,), (1,)), ((), ())), preferred_element_type=jnp.float32
    )
    h = (_silu(h1) * h3).astype(jnp.bfloat16)
    out = jax.lax.dot_general(
        h, cw, (((1,), (1,)), ((), ())), preferred_element_type=jnp.float32
    )
    eo_ref[0] = _pack_rows(out)


def _experts(xg, w1, w3, w2):
    return pl.pallas_call(
        _experts_body,
        grid=(E,),
        in_specs=[
            pl.BlockSpec((1, CAP, DH), lambda e: (e, 0, 0)),
            pl.BlockSpec((1, DF, D), lambda e: (e, 0, 0)),
            pl.BlockSpec((1, DF, D), lambda e: (e, 0, 0)),
            pl.BlockSpec((1, D, DF), lambda e: (e, 0, 0)),
        ],
        out_specs=pl.BlockSpec((1, CAP, DH), lambda e: (e, 0, 0)),
        out_shape=jax.ShapeDtypeStruct((E, CAP, DH), jnp.int32),
        compiler_params=pltpu.CompilerParams(
            dimension_semantics=("arbitrary",),
        ),
    )(xg.reshape(E, CAP, DH), w1, w3, w2)


_SB = 8


def _shared_body(xp_ref, sw1_ref, sw3_ref, sw2_ref, sh_ref):
    xs = _unpack_rows(xp_ref[...])
    sa = sw1_ref[...].astype(jnp.bfloat16)
    sb = sw3_ref[...].astype(jnp.bfloat16)
    sc = sw2_ref[...].astype(jnp.bfloat16)
    sh1 = jax.lax.dot_general(
        xs, sa, (((1,), (1,)), ((), ())), preferred_element_type=jnp.float32
    )
    sh3 = jax.lax.dot_general(
        xs, sb, (((1,), (1,)), ((), ())), preferred_element_type=jnp.float32
    )
    hs = (_silu(sh1) * sh3).astype(jnp.bfloat16)
    out = jax.lax.dot_general(
        hs, sc, (((1,), (1,)), ((), ())), preferred_element_type=jnp.float32
    )
    sh_ref[...] = _pack_rows(out)


def _shared(xp, sw1, sw3, sw2):
    tb = T // _SB
    return pl.pallas_call(
        _shared_body,
        grid=(_SB,),
        in_specs=[
            pl.BlockSpec((tb, DH), lambda i: (i, 0)),
            pl.BlockSpec((2 * DF, D), lambda i: (0, 0)),
            pl.BlockSpec((2 * DF, D), lambda i: (0, 0)),
            pl.BlockSpec((D, 2 * DF), lambda i: (0, 0)),
        ],
        out_specs=pl.BlockSpec((tb, DH), lambda i: (i, 0)),
        out_shape=jax.ShapeDtypeStruct((T, DH), jnp.int32),
        compiler_params=pltpu.CompilerParams(
            dimension_semantics=("arbitrary",),
        ),
    )(xp, sw1, sw3, sw2)


def _combine_body(ri_hbm, rf_hbm, eo_hbm, sh_hbm, y_hbm,
                  riv, rfv, idx0, idx1, g0, g1, shb, sem0, sem1, sem2):
    wid = lax.axis_index("s") * 2 + lax.axis_index("c")
    tb = wid * TPT
    pltpu.sync_copy(ri_hbm.at[pl.ds(tb * E, TPT * E)], riv)
    pltpu.sync_copy(rf_hbm.at[pl.ds(tb * E, TPT * E)], rfv)
    lanes = jax.lax.broadcasted_iota(jnp.int32, (L,), 0)
    for c in range(TPT // L):
        flat = (c * L + lanes) * E
        idx0[pl.ds(c * L, L)] = plsc.load_gather(riv, [flat + 2])
        idx1[pl.ds(c * L, L)] = plsc.load_gather(riv, [flat + 3])
    cp0 = pltpu.async_copy(eo_hbm.at[idx0], g0, sem0)
    cp1 = pltpu.async_copy(eo_hbm.at[idx1], g1, sem1)
    cp2 = pltpu.async_copy(sh_hbm.at[pl.ds(tb, TPT)], shb, sem2)
    cp0.wait()
    cp1.wait()
    cp2.wait()

    def tok(t, carry):
        row = jnp.full((L,), t * E, jnp.int32)
        w0 = plsc.load_gather(rfv, [row])
        w1 = plsc.load_gather(rfv, [row + 1])
        w0b = plsc.pack(w0, w0, format=plsc.PackFormat.INTERLEAVED)
        w1b = plsc.pack(w1, w1, format=plsc.PackFormat.INTERLEAVED)
        for j in range(DH // L):
            va = plsc.bitcast(g0[t, pl.ds(j * L, L)], jnp.bfloat16)
            vb = plsc.bitcast(g1[t, pl.ds(j * L, L)], jnp.bfloat16)
            vs = plsc.bitcast(shb[t, pl.ds(j * L, L)], jnp.bfloat16)
            acc = vs + w0b * va + w1b * vb
            shb[t, pl.ds(j * L, L)] = plsc.bitcast(acc, jnp.int32)
        return carry

    lax.fori_loop(0, TPT, tok, 0)
    pltpu.sync_copy(shb, y_hbm.at[pl.ds(tb, TPT)])


def _combine(ri, rf, eo, sh):
    mesh = plsc.VectorSubcoreMesh(core_axis_name="c", subcore_axis_name="s")
    f = pl.kernel(
        _combine_body,
        out_type=jax.ShapeDtypeStruct((T, DH), jnp.int32),
        mesh=mesh,
        scratch_types=[
            pltpu.VMEM((TPT * E,), jnp.int32),
            pltpu.VMEM((TPT * E,), jnp.float32),
            pltpu.VMEM((TPT,), jnp.int32),
            pltpu.VMEM((TPT,), jnp.int32),
            pltpu.VMEM((TPT, DH), jnp.int32),
            pltpu.VMEM((TPT, DH), jnp.int32),
            pltpu.VMEM((TPT, DH), jnp.int32),
            pltpu.SemaphoreType.DMA,
            pltpu.SemaphoreType.DMA,
            pltpu.SemaphoreType.DMA,
        ],
        compiler_params=pltpu.CompilerParams(needs_layout_passes=False),
    )
    return f(ri.reshape(T * E), rf.reshape(T * E), eo, sh)


def kernel(hidden_states, gate_w, w1, w3, w2, sw1, sw3, sw2):
    orig_shape = hidden_states.shape
    x = hidden_states.reshape(T, D).astype(jnp.float32)
    ri, rf, xp = _routing(x, gate_w.astype(jnp.float32))
    sh = _shared(xp, sw1, sw3, sw2)
    xg = _dispatch(ri, xp)
    eo = _experts(xg, w1, w3, w2)
    yp = _combine(ri, rf, eo.reshape(S, DH), sh)
    ub = jax.lax.bitcast_convert_type(yp, jnp.uint32)
    lo = jax.lax.bitcast_convert_type(ub << 16, jnp.float32)
    hi = jax.lax.bitcast_convert_type(ub & jnp.uint32(0xFFFF0000), jnp.float32)
    y = jnp.concatenate([lo, hi], axis=1)
    return y.reshape(orig_shape)


# speedup vs baseline: 1.0713x; 1.0001x over previous
import functools

import jax
import jax.numpy as jnp
from jax import lax
from jax.experimental import pallas as pl
from jax.experimental.pallas import tpu as pltpu
from jax.experimental.pallas import tpu_sc as plsc

T = 2048
D = 1024
DH = D // 2
E = 8
DF = 512
CAP = 512
S = E * CAP
NW = 32
SPT = S // NW
TPT = T // NW
L = 16
_NEG = -1e30


def _routing_body(x_ref, gw_ref, ri_ref, rf_ref, xp_ref):
    x = x_ref[...]
    gw = gw_ref[...]
    xp_ref[...] = _pack_rows(x)
    logits = jax.lax.dot_general(
        x, gw, (((1,), (1,)), ((), ())), preferred_element_type=jnp.float32
    )
    m = jnp.max(logits, axis=-1, keepdims=True)
    ex = jnp.exp(logits - m)
    scores = ex / jnp.sum(ex, axis=-1, keepdims=True)
    eidx = jax.lax.broadcasted_iota(jnp.int32, (T, E), 1)
    s0 = jnp.max(scores, axis=-1, keepdims=True)
    i0 = jnp.min(jnp.where(scores >= s0, eidx, E), axis=-1, keepdims=True)
    oh0 = eidx == i0
    sc1 = jnp.where(oh0, _NEG, scores)
    s1 = jnp.max(sc1, axis=-1, keepdims=True)
    i1 = jnp.min(jnp.where(sc1 >= s1, eidx, E), axis=-1, keepdims=True)
    oh1 = eidx == i1
    cnt = oh0.astype(jnp.bfloat16) + oh1.astype(jnp.bfloat16)
    nb = 8
    tb = T // nb
    r = jax.lax.broadcasted_iota(jnp.int32, (tb, tb), 0)
    c = jax.lax.broadcasted_iota(jnp.int32, (tb, tb), 1)
    lmask = (c < r).astype(jnp.bfloat16)
    ones = jnp.ones((1, tb), jnp.bfloat16)
    parts = []
    run = jnp.zeros((1, E), jnp.float32)
    for b in range(nb):
        cb = cnt[b * tb:(b + 1) * tb, :]
        intra = jax.lax.dot_general(
            lmask, cb, (((1,), (0,)), ((), ())), preferred_element_type=jnp.float32
        )
        parts.append(intra + run)
        tot = jax.lax.dot_general(
            ones, cb, (((1,), (0,)), ((), ())), preferred_element_type=jnp.float32
        )
        run = run + tot
    cum = jnp.concatenate(parts, axis=0)
    pos0 = jnp.sum(jnp.where(oh0, cum, 0.0), axis=-1, keepdims=True).astype(jnp.int32)
    pos1 = jnp.sum(jnp.where(oh1, cum, 0.0), axis=-1, keepdims=True).astype(jnp.int32)
    v0 = pos0 < CAP
    v1 = pos1 < CAP
    denom = s0 + s1 + 1e-20
    w0 = jnp.where(v0, s0 / denom, 0.0)
    w1 = jnp.where(v1, s1 / denom, 0.0)
    d0 = i0 * CAP + pos0
    d1 = i1 * CAP + pos1
    d0m = jnp.where(v0, d0, -1)
    d1m = jnp.where(v1, d1, -1)
    d0c = jnp.where(v0, d0, 0)
    d1c = jnp.where(v1, d1, 0)
    zi = jnp.zeros((T, 1), jnp.int32)
    ri = jnp.concatenate([d0m, d1m, d0c, d1c, zi, zi, zi, zi], axis=1)
    zf = jnp.zeros((T, 1), jnp.float32)
    rf = jnp.concatenate([w0, w1, zf, zf, zf, zf, zf, zf], axis=1)
    ri_ref[...] = ri
    rf_ref[...] = rf


def _routing(x, gate_w):
    return pl.pallas_call(
        _routing_body,
        out_shape=(
            jax.ShapeDtypeStruct((T, E), jnp.int32),
            jax.ShapeDtypeStruct((T, E), jnp.float32),
            jax.ShapeDtypeStruct((T, DH), jnp.int32),
        ),
    )(x, gate_w)


def _dispatch_body(ri_hbm, xp_hbm, xg_hbm, riv, src, rows, sem):
    wid = lax.axis_index("s") * 2 + lax.axis_index("c")
    base = wid * SPT
    pltpu.sync_copy(ri_hbm, riv)
    for k in range(SPT // L):
        src[pl.ds(k * L, L)] = jnp.zeros((L,), jnp.int32)
    lanes = jax.lax.broadcasted_iota(jnp.int32, (L,), 0)

    def chunk(c, carry):
        rowi = c * L + lanes
        flat = rowi * E
        d0 = plsc.load_gather(riv, [flat])
        d1 = plsc.load_gather(riv, [flat + 1])
        m0 = (d0 >= base) & (d0 < base + SPT)
        l0 = jnp.where(m0, d0 - base, 0)
        plsc.store_scatter(src, [l0], rowi, mask=m0)
        m1 = (d1 >= base) & (d1 < base + SPT)
        l1 = jnp.where(m1, d1 - base, 0)
        plsc.store_scatter(src, [l1], rowi, mask=m1)
        return carry

    lax.fori_loop(0, T // L, chunk, 0)
    half = SPT // 2
    for g in range(2):
        pltpu.async_copy(xp_hbm.at[src.at[pl.ds(g * half, half)]], rows, sem).wait()
        pltpu.sync_copy(rows, xg_hbm.at[pl.ds(base + g * half, half)])


def _dispatch(ri, xp):
    mesh = plsc.VectorSubcoreMesh(core_axis_name="c", subcore_axis_name="s")
    half = SPT // 2
    f = pl.kernel(
        _dispatch_body,
        out_type=jax.ShapeDtypeStruct((S, DH), jnp.int32),
        mesh=mesh,
        scratch_types=[
            pltpu.VMEM((T * E,), jnp.int32),
            pltpu.VMEM((SPT,), jnp.int32),
            pltpu.VMEM((half, DH), jnp.int32),
            pltpu.SemaphoreType.DMA,
        ],
        compiler_params=pltpu.CompilerParams(needs_layout_passes=False, skip_device_barrier=True),
    )
    return f(ri.reshape(T * E), xp)


def _silu(h):
    return h / (1.0 + jnp.exp(-h))


def _pack_rows(r):
    rb = r.astype(jnp.bfloat16).astype(jnp.float32)
    ub = jax.lax.bitcast_convert_type(rb, jnp.uint32)
    lo = ub[:, :DH] >> 16
    hi = ub[:, DH:] & jnp.uint32(0xFFFF0000)
    return jax.lax.bitcast_convert_type(hi | lo, jnp.int32)


def _unpack_rows(p):
    ub = jax.lax.bitcast_convert_type(p, jnp.uint32)
    lo = jax.lax.bitcast_convert_type(ub << 16, jnp.float32)
    hi = jax.lax.bitcast_convert_type(ub & jnp.uint32(0xFFFF0000), jnp.float32)
    return jnp.concatenate([lo, hi], axis=1).astype(jnp.bfloat16)


def _experts_body(xg_ref, w1_ref, w3_ref, w2_ref, eo_ref):
    xgb = _unpack_rows(xg_ref[0])
    a = w1_ref[0].astype(jnp.bfloat16)
    b = w3_ref[0].astype(jnp.bfloat16)
    cw = w2_ref[0].astype(jnp.bfloat16)
    h1 = jax.lax.dot_general(
        xgb, a, (((1,), (1,)), ((), ())), preferred_element_type=jnp.float32
    )
    h3 = jax.lax.dot_general(
        xgb, b, (((1,), (1,)), ((), ())), preferred_element_type=jnp.float32
    )
    h = (_silu(h1) * h3).astype(jnp.bfloat16)
    out = jax.lax.dot_general(
        h, cw, (((1,), (1,)), ((), ())), preferred_element_type=jnp.float32
    )
    eo_ref[0] = _pack_rows(out)


def _experts(xg, w1, w3, w2):
    return pl.pallas_call(
        _experts_body,
        grid=(E,),
        in_specs=[
            pl.BlockSpec((1, CAP, DH), lambda e: (e, 0, 0)),
            pl.BlockSpec((1, DF, D), lambda e: (e, 0, 0)),
            pl.BlockSpec((1, DF, D), lambda e: (e, 0, 0)),
            pl.BlockSpec((1, D, DF), lambda e: (e, 0, 0)),
        ],
        out_specs=pl.BlockSpec((1, CAP, DH), lambda e: (e, 0, 0)),
        out_shape=jax.ShapeDtypeStruct((E, CAP, DH), jnp.int32),
        compiler_params=pltpu.CompilerParams(
            dimension_semantics=("arbitrary",),
        ),
    )(xg.reshape(E, CAP, DH), w1, w3, w2)


_SB = 8


def _shared_body(xp_ref, sw1_ref, sw3_ref, sw2_ref, sh_ref):
    xs = _unpack_rows(xp_ref[...])
    sa = sw1_ref[...].astype(jnp.bfloat16)
    sb = sw3_ref[...].astype(jnp.bfloat16)
    sc = sw2_ref[...].astype(jnp.bfloat16)
    sh1 = jax.lax.dot_general(
        xs, sa, (((1,), (1,)), ((), ())), preferred_element_type=jnp.float32
    )
    sh3 = jax.lax.dot_general(
        xs, sb, (((1,), (1,)), ((), ())), preferred_element_type=jnp.float32
    )
    hs = (_silu(sh1) * sh3).astype(jnp.bfloat16)
    out = jax.lax.dot_general(
        hs, sc, (((1,), (1,)), ((), ())), preferred_element_type=jnp.float32
    )
    sh_ref[...] = _pack_rows(out)


def _shared(xp, sw1, sw3, sw2):
    tb = T // _SB
    return pl.pallas_call(
        _shared_body,
        grid=(_SB,),
        in_specs=[
            pl.BlockSpec((tb, DH), lambda i: (i, 0)),
            pl.BlockSpec((2 * DF, D), lambda i: (0, 0)),
            pl.BlockSpec((2 * DF, D), lambda i: (0, 0)),
            pl.BlockSpec((D, 2 * DF), lambda i: (0, 0)),
        ],
        out_specs=pl.BlockSpec((tb, DH), lambda i: (i, 0)),
        out_shape=jax.ShapeDtypeStruct((T, DH), jnp.int32),
        compiler_params=pltpu.CompilerParams(
            dimension_semantics=("arbitrary",),
        ),
    )(xp, sw1, sw3, sw2)


def _combine_body(ri_hbm, rf_hbm, eo_hbm, sh_hbm, y_hbm,
                  riv, rfv, idx0, idx1, g0, g1, shb, sem0, sem1, sem2):
    wid = lax.axis_index("s") * 2 + lax.axis_index("c")
    tb = wid * TPT
    pltpu.sync_copy(ri_hbm.at[pl.ds(tb * E, TPT * E)], riv)
    pltpu.sync_copy(rf_hbm.at[pl.ds(tb * E, TPT * E)], rfv)
    lanes = jax.lax.broadcasted_iota(jnp.int32, (L,), 0)
    for c in range(TPT // L):
        flat = (c * L + lanes) * E
        idx0[pl.ds(c * L, L)] = plsc.load_gather(riv, [flat + 2])
        idx1[pl.ds(c * L, L)] = plsc.load_gather(riv, [flat + 3])
    cp0 = pltpu.async_copy(eo_hbm.at[idx0], g0, sem0)
    cp1 = pltpu.async_copy(eo_hbm.at[idx1], g1, sem1)
    cp2 = pltpu.async_copy(sh_hbm.at[pl.ds(tb, TPT)], shb, sem2)
    cp0.wait()
    cp1.wait()
    cp2.wait()

    def tok(t, carry):
        row = jnp.full((L,), t * E, jnp.int32)
        w0 = plsc.load_gather(rfv, [row])
        w1 = plsc.load_gather(rfv, [row + 1])
        w0b = plsc.pack(w0, w0, format=plsc.PackFormat.INTERLEAVED)
        w1b = plsc.pack(w1, w1, format=plsc.PackFormat.INTERLEAVED)
        for j in range(DH // L):
            va = plsc.bitcast(g0[t, pl.ds(j * L, L)], jnp.bfloat16)
            vb = plsc.bitcast(g1[t, pl.ds(j * L, L)], jnp.bfloat16)
            vs = plsc.bitcast(shb[t, pl.ds(j * L, L)], jnp.bfloat16)
            acc = vs + w0b * va + w1b * vb
            shb[t, pl.ds(j * L, L)] = plsc.bitcast(acc, jnp.int32)
        return carry

    lax.fori_loop(0, TPT, tok, 0)
    pltpu.sync_copy(shb, y_hbm.at[pl.ds(tb, TPT)])


def _combine(ri, rf, eo, sh):
    mesh = plsc.VectorSubcoreMesh(core_axis_name="c", subcore_axis_name="s")
    f = pl.kernel(
        _combine_body,
        out_type=jax.ShapeDtypeStruct((T, DH), jnp.int32),
        mesh=mesh,
        scratch_types=[
            pltpu.VMEM((TPT * E,), jnp.int32),
            pltpu.VMEM((TPT * E,), jnp.float32),
            pltpu.VMEM((TPT,), jnp.int32),
            pltpu.VMEM((TPT,), jnp.int32),
            pltpu.VMEM((TPT, DH), jnp.int32),
            pltpu.VMEM((TPT, DH), jnp.int32),
            pltpu.VMEM((TPT, DH), jnp.int32),
            pltpu.SemaphoreType.DMA,
            pltpu.SemaphoreType.DMA,
            pltpu.SemaphoreType.DMA,
        ],
        compiler_params=pltpu.CompilerParams(needs_layout_passes=False, skip_device_barrier=True),
    )
    return f(ri.reshape(T * E), rf.reshape(T * E), eo, sh)


def kernel(hidden_states, gate_w, w1, w3, w2, sw1, sw3, sw2):
    orig_shape = hidden_states.shape
    x = hidden_states.reshape(T, D).astype(jnp.float32)
    ri, rf, xp = _routing(x, gate_w.astype(jnp.float32))
    sh = _shared(xp, sw1, sw3, sw2)
    xg = _dispatch(ri, xp)
    eo = _experts(xg, w1, w3, w2)
    yp = _combine(ri, rf, eo.reshape(S, DH), sh)
    ub = jax.lax.bitcast_convert_type(yp, jnp.uint32)
    lo = jax.lax.bitcast_convert_type(ub << 16, jnp.float32)
    hi = jax.lax.bitcast_convert_type(ub & jnp.uint32(0xFFFF0000), jnp.float32)
    y = jnp.concatenate([lo, hi], axis=1)
    return y.reshape(orig_shape)
